# Initial kernel scaffold; baseline (speedup 1.0000x reference)
import jax
import jax.numpy as jnp
from jax.experimental import pallas as pl


def kernel(relation_ids, embedding):
    raise NotImplementedError("write your pallas kernel here")


# speedup vs baseline: 1.2870x; 1.2870x over previous
import functools

import jax
import jax.numpy as jnp
from jax import lax
from jax.experimental import pallas as pl
from jax.experimental.pallas import tpu as pltpu
from jax.experimental.pallas import tpu_sc as plsc

_NUM_CORES = 2
_NUM_SUBCORES = 16
_NUM_WORKERS = _NUM_CORES * _NUM_SUBCORES

_CHUNK = 64


@functools.partial(jax.jit, static_argnames=("rows_per_worker",))
def _sc_gather(embedding, flat_ids, rows_per_worker):
    num_rows, d = flat_ids.shape[0], embedding.shape[1]
    num_chunks = rows_per_worker // _CHUNK
    mesh = plsc.VectorSubcoreMesh(
        core_axis_name="c",
        subcore_axis_name="s",
        num_cores=_NUM_CORES,
        num_subcores=_NUM_SUBCORES,
    )

    @functools.partial(
        pl.kernel,
        out_type=jax.ShapeDtypeStruct((num_rows, d), jnp.float32),
        mesh=mesh,
        scratch_types=[
            pltpu.VMEM((rows_per_worker,), jnp.int32),
            pltpu.VMEM((_CHUNK, d), jnp.float32),
            pltpu.SemaphoreType.DMA,
        ],
    )
    def k(table_hbm, idx_hbm, out_hbm, idx_v, rows_v, sem):
        wid = lax.axis_index("s") * _NUM_CORES + lax.axis_index("c")
        base = wid * rows_per_worker
        pltpu.sync_copy(idx_hbm.at[pl.ds(base, rows_per_worker)], idx_v)

        def body(c, _):
            idx_chunk = idx_v.at[pl.ds(c * _CHUNK, _CHUNK)]
            pltpu.async_copy(table_hbm.at[idx_chunk], rows_v, sem).wait()
            pltpu.sync_copy(rows_v, out_hbm.at[pl.ds(base + c * _CHUNK, _CHUNK)])
            return _

        lax.fori_loop(0, num_chunks, body, None)

    return k(embedding, flat_ids)


def kernel(relation_ids, embedding):
    b, n = relation_ids.shape
    d = embedding.shape[1]
    num_rows = b * n
    assert num_rows % (_NUM_WORKERS * _CHUNK) == 0
    flat_ids = relation_ids.reshape(-1).astype(jnp.int32)
    out = _sc_gather(embedding.astype(jnp.float32), flat_ids,
                     num_rows // _NUM_WORKERS)
    return out.reshape(b, n, d)


# speedup vs baseline: 1.2899x; 1.0023x over previous
import functools

import jax
import jax.numpy as jnp
from jax import lax
from jax.experimental import pallas as pl
from jax.experimental.pallas import tpu as pltpu
from jax.experimental.pallas import tpu_sc as plsc

_NUM_CORES = 2
_NUM_SUBCORES = 16
_NUM_WORKERS = _NUM_CORES * _NUM_SUBCORES

_CHUNK = 80


@functools.partial(jax.jit, static_argnames=("rows_per_worker",))
def _sc_gather(embedding, flat_ids, rows_per_worker):
    num_rows, d = flat_ids.shape[0], embedding.shape[1]
    num_chunks = rows_per_worker // _CHUNK
    num_groups = num_chunks // 2
    mesh = plsc.VectorSubcoreMesh(
        core_axis_name="c",
        subcore_axis_name="s",
        num_cores=_NUM_CORES,
        num_subcores=_NUM_SUBCORES,
    )

    @functools.partial(
        pl.kernel,
        out_type=jax.ShapeDtypeStruct((num_rows, d), jnp.float32),
        mesh=mesh,
        scratch_types=[
            pltpu.VMEM((rows_per_worker,), jnp.int32),
            pltpu.VMEM((2, _CHUNK, d), jnp.float32),
            pltpu.SemaphoreType.DMA,
            pltpu.SemaphoreType.DMA,
        ],
    )
    def k(table_hbm, idx_hbm, out_hbm, idx_v, buf_v, gsem0, gsem1):
        gsems = (gsem0, gsem1)
        wid = lax.axis_index("s") * _NUM_CORES + lax.axis_index("c")
        base = wid * rows_per_worker
        pltpu.sync_copy(idx_hbm.at[pl.ds(base, rows_per_worker)], idx_v)

        def start_gather(c, b):
            idx_chunk = idx_v.at[pl.ds(c * _CHUNK, _CHUNK)]
            pltpu.async_copy(table_hbm.at[idx_chunk], buf_v.at[b], gsems[b])

        def wait_gather(b):
            pltpu.make_async_copy(
                table_hbm.at[pl.ds(0, _CHUNK)], buf_v.at[b], gsems[b]).wait()

        def scatter(c, b):
            pltpu.sync_copy(buf_v.at[b], out_hbm.at[pl.ds(base + c * _CHUNK, _CHUNK)])

        start_gather(0, 0)

        def body(g, _):
            c = 2 * g
            start_gather(c + 1, 1)
            wait_gather(0)
            scatter(c, 0)
            start_gather(c + 2, 0)
            wait_gather(1)
            scatter(c + 1, 1)
            return _

        lax.fori_loop(0, num_groups - 1, body, None)

        c = 2 * (num_groups - 1)
        start_gather(c + 1, 1)
        wait_gather(0)
        scatter(c, 0)
        wait_gather(1)
        scatter(c + 1, 1)

    return k(embedding, flat_ids)


def kernel(relation_ids, embedding):
    b, n = relation_ids.shape
    d = embedding.shape[1]
    num_rows = b * n
    assert num_rows % (_NUM_WORKERS * _CHUNK) == 0
    flat_ids = relation_ids.reshape(-1).astype(jnp.int32)
    out = _sc_gather(embedding.astype(jnp.float32), flat_ids,
                     num_rows // _NUM_WORKERS)
    return out.reshape(b, n, d)


# speedup vs baseline: 1.2900x; 1.0001x over previous
import functools

import jax
import jax.numpy as jnp
from jax import lax
from jax.experimental import pallas as pl
from jax.experimental.pallas import tpu as pltpu
from jax.experimental.pallas import tpu_sc as plsc

_NUM_CORES = 2
_NUM_SUBCORES = 16
_NUM_WORKERS = _NUM_CORES * _NUM_SUBCORES

_CHUNK = 64


@functools.partial(jax.jit, static_argnames=("rows_per_worker",))
def _sc_gather(embedding, flat_ids, rows_per_worker):
    num_rows, d = flat_ids.shape[0], embedding.shape[1]
    num_chunks = rows_per_worker // _CHUNK
    num_groups = num_chunks // 2
    mesh = plsc.VectorSubcoreMesh(
        core_axis_name="c",
        subcore_axis_name="s",
        num_cores=_NUM_CORES,
        num_subcores=_NUM_SUBCORES,
    )

    @functools.partial(
        pl.kernel,
        out_type=jax.ShapeDtypeStruct((num_rows, d), jnp.float32),
        mesh=mesh,
        scratch_types=[
            pltpu.VMEM((rows_per_worker,), jnp.int32),
            pltpu.VMEM((2, _CHUNK, d), jnp.float32),
            pltpu.SemaphoreType.DMA,
            pltpu.SemaphoreType.DMA,
        ],
    )
    def k(table_hbm, idx_hbm, out_hbm, idx_v, buf_v, gsem0, gsem1):
        gsems = (gsem0, gsem1)
        wid = lax.axis_index("s") * _NUM_CORES + lax.axis_index("c")
        base = wid * rows_per_worker

        pltpu.sync_copy(idx_hbm.at[pl.ds(base, rows_per_worker)], idx_v)

        def start_gather(c, b):
            idx_chunk = idx_v.at[pl.ds(c * _CHUNK, _CHUNK)]
            pltpu.async_copy(table_hbm.at[idx_chunk], buf_v.at[b], gsems[b])

        def wait_gather(b):
            pltpu.make_async_copy(
                table_hbm.at[pl.ds(0, _CHUNK)], buf_v.at[b], gsems[b]).wait()

        def scatter(c, b):
            pltpu.sync_copy(buf_v.at[b], out_hbm.at[pl.ds(base + c * _CHUNK, _CHUNK)])

        start_gather(0, 0)

        def body(g, _):
            c = 2 * g
            start_gather(c + 1, 1)
            wait_gather(0)
            scatter(c, 0)
            start_gather(c + 2, 0)
            wait_gather(1)
            scatter(c + 1, 1)
            return _

        lax.fori_loop(0, num_groups - 1, body, None)

        c = 2 * (num_groups - 1)
        start_gather(c + 1, 1)
        wait_gather(0)
        scatter(c, 0)
        wait_gather(1)
        scatter(c + 1, 1)

    return k(embedding, flat_ids)


def kernel(relation_ids, embedding):
    b, n = relation_ids.shape
    d = embedding.shape[1]
    num_rows = b * n
    assert num_rows % (_NUM_WORKERS * _CHUNK) == 0
    flat_ids = relation_ids.reshape(-1).astype(jnp.int32)
    out = _sc_gather(embedding.astype(jnp.float32), flat_ids,
                     num_rows // _NUM_WORKERS)
    return out.reshape(b, n, d)
